# baseline (speedup 1.0000x reference)
import jax
import jax.numpy as jnp
from jax import lax
from jax.experimental import pallas as pl
from jax.experimental.pallas import tpu as pltpu
from jax.experimental.pallas import tpu_sc as plsc

NUM_FACTORS = 32
BATCH = 16384
NC, NS, L = 2, 16, 16
NW = NC * NS
B_PER_W = BATCH // NW
PANEL = 128
NBUF = 12
NCHUNK = B_PER_W // NBUF
TAIL = B_PER_W - NCHUNK * NBUF


def _body(utabT, itabT, u_idx, i_idx, w_hbm, b_hbm, out_hbm,
          ubuf, ibuf, wv, bv, outv, idxu, idxi, us_smem, is_smem, *sems):
    wid = lax.axis_index("s") * NC + lax.axis_index("c")
    base = wid * B_PER_W

    pltpu.sync_copy(u_idx.at[pl.ds(base, B_PER_W)], idxu)
    pltpu.sync_copy(i_idx.at[pl.ds(base, B_PER_W)], idxi)
    pltpu.sync_copy(w_hbm, wv)
    pltpu.sync_copy(b_hbm, bv)

    dlo = jnp.arange(L, dtype=jnp.int32)

    def unpack(g, _):
        uvec = idxu[pl.ds(g * L, L)]
        ivec = idxi[pl.ds(g * L, L)]
        for k in range(L):
            us_smem[g * L + k] = jnp.sum(jnp.where(dlo == k, uvec, 0))
            is_smem[g * L + k] = jnp.sum(jnp.where(dlo == k, ivec, 0))
        return 0

    lax.fori_loop(0, B_PER_W // L, unpack, 0)

    def fetch(b, slot):
        u = us_smem[b]
        i = is_smem[b]
        cu = pl.multiple_of((u >> 7) * PANEL, PANEL)
        ci = pl.multiple_of((i >> 7) * PANEL, PANEL)
        pltpu.async_copy(utabT.at[:, pl.ds(cu, PANEL)], ubuf.at[slot], sems[slot])
        pltpu.async_copy(itabT.at[:, pl.ds(ci, PANEL)], ibuf.at[slot], sems[slot])

    w0 = wv[pl.ds(0, L)]
    w1 = wv[pl.ds(L, L)]
    bias = bv[...]

    for j in range(NBUF):
        fetch(j, j)

    def element(b, slot, acc):
        pltpu.make_async_copy(
            utabT.at[:, pl.ds(0, PANEL)], ubuf.at[slot], sems[slot]).wait()
        pltpu.make_async_copy(
            itabT.at[:, pl.ds(0, PANEL)], ibuf.at[slot], sems[slot]).wait()

        @pl.when(b + NBUF < B_PER_W)
        def _():
            fetch(b + NBUF, slot)

        u = us_smem[b]
        i = is_smem[b]
        ul = jnp.full((L,), u & (PANEL - 1), dtype=jnp.int32)
        il = jnp.full((L,), i & (PANEL - 1), dtype=jnp.int32)
        u0 = plsc.load_gather(ubuf.at[slot], [dlo, ul])
        u1 = plsc.load_gather(ubuf.at[slot], [dlo + L, ul])
        i0 = plsc.load_gather(ibuf.at[slot], [dlo, il])
        i1 = plsc.load_gather(ibuf.at[slot], [dlo + L, il])
        s = jnp.sum(u0 * i0 * w0 + u1 * i1 * w1)
        acc = jnp.where(dlo == (b & (L - 1)), s, acc)

        @pl.when((b & (L - 1)) == L - 1)
        def _():
            outv[pl.ds((b >> 4) << 4, L)] = \
                1.0 / (1.0 + jnp.exp(-(acc + bias)))

        return acc

    def chunk(c, acc):
        for j in range(NBUF):
            acc = element(c * NBUF + j, j, acc)
        return acc

    acc = lax.fori_loop(0, NCHUNK, chunk, jnp.zeros((L,), jnp.float32))
    for j in range(TAIL):
        acc = element(NCHUNK * NBUF + j, j, acc)

    pltpu.sync_copy(outv, out_hbm.at[pl.ds(base, B_PER_W)])


@jax.jit
def _gmf(utabT, itabT, users, items, w, b16):
    mesh = plsc.VectorSubcoreMesh(core_axis_name="c", subcore_axis_name="s")
    return pl.kernel(
        _body,
        out_type=jax.ShapeDtypeStruct((BATCH,), jnp.float32),
        mesh=mesh,
        compiler_params=pltpu.CompilerParams(
            needs_layout_passes=False, use_tc_tiling_on_sc=True),
        scratch_types=[
            pltpu.VMEM((NBUF, NUM_FACTORS, PANEL), jnp.float32),
            pltpu.VMEM((NBUF, NUM_FACTORS, PANEL), jnp.float32),
            pltpu.VMEM((NUM_FACTORS,), jnp.float32),
            pltpu.VMEM((L,), jnp.float32),
            pltpu.VMEM((B_PER_W,), jnp.float32),
            pltpu.VMEM((B_PER_W,), jnp.int32),
            pltpu.VMEM((B_PER_W,), jnp.int32),
            pltpu.SMEM((B_PER_W,), jnp.int32),
            pltpu.SMEM((B_PER_W,), jnp.int32),
        ] + [pltpu.SemaphoreType.DMA] * NBUF,
    )(utabT, itabT, users, items, w, b16)


def kernel(users, items, user_table, item_table, fc_w, fc_b):
    utabT = user_table.T
    itabT = item_table.T
    w = fc_w.reshape(NUM_FACTORS)
    b16 = jnp.broadcast_to(fc_b.reshape(1), (L,))
    return _gmf(utabT, itabT, users.astype(jnp.int32), items.astype(jnp.int32),
                w, b16)


# speedup vs baseline: 1.0186x; 1.0186x over previous
import jax
import jax.numpy as jnp
from jax import lax
from jax.experimental import pallas as pl
from jax.experimental.pallas import tpu as pltpu
from jax.experimental.pallas import tpu_sc as plsc

NUM_FACTORS = 32
BATCH = 16384
NC, NS, L = 2, 16, 16
NW = NC * NS
B_PER_W = BATCH // NW
PANEL = 128
NBUF = 8
NCHUNK = B_PER_W // NBUF


def _body(utabT, itabT, u_idx, i_idx, w_hbm, b_hbm, out_hbm,
          ubuf, ibuf, wv, bv, outv, idxu, idxi, *sems):
    wid = lax.axis_index("s") * NC + lax.axis_index("c")
    base = wid * B_PER_W

    pltpu.sync_copy(u_idx.at[pl.ds(base, B_PER_W)], idxu)
    pltpu.sync_copy(i_idx.at[pl.ds(base, B_PER_W)], idxi)
    pltpu.sync_copy(w_hbm, wv)
    pltpu.sync_copy(b_hbm, bv)

    dlo = jnp.arange(L, dtype=jnp.int32)

    def extract(vec, k):
        return jnp.sum(jnp.where(dlo == k, vec, 0))

    def fetch(uvec, ivec, k, slot):
        u = extract(uvec, k)
        i = extract(ivec, k)
        cu = pl.multiple_of((u >> 7) * PANEL, PANEL)
        ci = pl.multiple_of((i >> 7) * PANEL, PANEL)
        pltpu.async_copy(utabT.at[:, pl.ds(cu, PANEL)], ubuf.at[slot], sems[slot])
        pltpu.async_copy(itabT.at[:, pl.ds(ci, PANEL)], ibuf.at[slot], sems[slot])

    w0 = wv[pl.ds(0, L)]
    w1 = wv[pl.ds(L, L)]
    bias = bv[...]

    uvec0 = idxu[pl.ds(0, L)]
    ivec0 = idxi[pl.ds(0, L)]
    for j in range(NBUF):
        fetch(uvec0, ivec0, j, j)

    def chunk(c, acc):
        vb = pl.multiple_of(((c * NBUF) >> 4) * L, L)
        uvec = idxu[pl.ds(vb, L)]
        ivec = idxi[pl.ds(vb, L)]
        vb_n = pl.multiple_of((((c + 1) * NBUF) >> 4) * L, L)
        uvec_n = idxu[pl.ds(vb_n, L)]
        ivec_n = idxi[pl.ds(vb_n, L)]
        half = (c * NBUF) & (L - 1)
        half_n = ((c + 1) * NBUF) & (L - 1)
        for j in range(NBUF):
            b = c * NBUF + j
            pltpu.make_async_copy(
                utabT.at[:, pl.ds(0, PANEL)], ubuf.at[j], sems[j]).wait()
            pltpu.make_async_copy(
                itabT.at[:, pl.ds(0, PANEL)], ibuf.at[j], sems[j]).wait()
            u = extract(uvec, half + j)
            i = extract(ivec, half + j)
            ul = jnp.full((L,), u & (PANEL - 1), dtype=jnp.int32)
            il = jnp.full((L,), i & (PANEL - 1), dtype=jnp.int32)
            u0 = plsc.load_gather(ubuf.at[j], [dlo, ul])
            u1 = plsc.load_gather(ubuf.at[j], [dlo + L, ul])
            i0 = plsc.load_gather(ibuf.at[j], [dlo, il])
            i1 = plsc.load_gather(ibuf.at[j], [dlo + L, il])

            @pl.when(c + 1 < NCHUNK)
            def _():
                fetch(uvec_n, ivec_n, half_n + j, j)

            s = jnp.sum(u0 * i0 * w0 + u1 * i1 * w1)
            acc = jnp.where(dlo == (b & (L - 1)), s, acc)

            @pl.when((b & (L - 1)) == L - 1)
            def _():
                outv[pl.ds((b >> 4) << 4, L)] = \
                    1.0 / (1.0 + jnp.exp(-(acc + bias)))
        return acc

    lax.fori_loop(0, NCHUNK, chunk, jnp.zeros((L,), jnp.float32))

    pltpu.sync_copy(outv, out_hbm.at[pl.ds(base, B_PER_W)])


@jax.jit
def _gmf(utabT, itabT, users, items, w, b16):
    mesh = plsc.VectorSubcoreMesh(core_axis_name="c", subcore_axis_name="s")
    return pl.kernel(
        _body,
        out_type=jax.ShapeDtypeStruct((BATCH,), jnp.float32),
        mesh=mesh,
        compiler_params=pltpu.CompilerParams(
            needs_layout_passes=False, use_tc_tiling_on_sc=True),
        scratch_types=[
            pltpu.VMEM((NBUF, NUM_FACTORS, PANEL), jnp.float32),
            pltpu.VMEM((NBUF, NUM_FACTORS, PANEL), jnp.float32),
            pltpu.VMEM((NUM_FACTORS,), jnp.float32),
            pltpu.VMEM((L,), jnp.float32),
            pltpu.VMEM((B_PER_W,), jnp.float32),
            pltpu.VMEM((B_PER_W,), jnp.int32),
            pltpu.VMEM((B_PER_W,), jnp.int32),
        ] + [pltpu.SemaphoreType.DMA] * NBUF,
    )(utabT, itabT, users, items, w, b16)


def kernel(users, items, user_table, item_table, fc_w, fc_b):
    utabT = user_table.T
    itabT = item_table.T
    w = fc_w.reshape(NUM_FACTORS)
    b16 = jnp.broadcast_to(fc_b.reshape(1), (L,))
    return _gmf(utabT, itabT, users.astype(jnp.int32), items.astype(jnp.int32),
                w, b16)
